# Initial kernel scaffold; baseline (speedup 1.0000x reference)
import jax
import jax.numpy as jnp
from jax.experimental import pallas as pl


def kernel(pos, batch, W1a, b1a, W1b, b1b, W2a, b2a, W2b, b2b):
    raise NotImplementedError("write your pallas kernel here")


# speedup vs baseline: 46.9253x; 46.9253x over previous
import functools

import jax
import jax.numpy as jnp
from jax import lax
from jax.experimental import pallas as pl
from jax.experimental.pallas import tpu as pltpu
from jax.experimental.pallas import tpu_sc as plsc

K = 16
B_Q = 400
C_T = 512
BIG_F = 1.0e9


def _knn_body(bounds_ref, qf_ref, cand_ref, nbr_ref):
    i = pl.program_id(0)
    t0 = bounds_ref[i, 0]
    t1 = bounds_ref[i, 1]
    qx = qf_ref[:, 0:1]
    qy = qf_ref[:, 1:2]
    qz = qf_ref[:, 2:3]
    qb = qf_ref[:, 3:4]
    B = qf_ref.shape[0]

    def tile_step(t, carry):
        bd, bi = carry
        cand = cand_ref[t]
        cx = cand[0:1, :]
        cy = cand[1:2, :]
        cz = cand[2:3, :]
        cb = cand[3:4, :]
        dx = qx - cx
        dy = qy - cy
        dz = qz - cz
        d = (dx * dx + dy * dy) + dz * dz
        d = jnp.where(qb == cb, d, jnp.inf)
        colg = (lax.broadcasted_iota(jnp.int32, (B, C_T), 1).astype(jnp.float32)
                + t.astype(jnp.float32) * float(C_T))
        cd = jnp.concatenate([bd, d], axis=1)
        ci = jnp.concatenate([bi, colg], axis=1)
        nds, nis = [], []
        for _ in range(K):
            dmin = jnp.min(cd, axis=1, keepdims=True)
            imin = jnp.min(jnp.where(cd == dmin, ci, BIG_F), axis=1,
                           keepdims=True)
            nds.append(dmin)
            nis.append(imin)
            cd = jnp.where(ci == imin, jnp.inf, cd)
        return jnp.concatenate(nds, axis=1), jnp.concatenate(nis, axis=1)

    bd0 = jnp.full((B, K), jnp.inf, jnp.float32)
    bi0 = jnp.full((B, K), BIG_F, jnp.float32)
    _, bi = lax.fori_loop(t0, t1, tile_step, (bd0, bi0))
    n_total = nbr_ref.shape[0] * pl.num_programs(0)
    nbr_ref[...] = jnp.clip(bi.astype(jnp.int32), 0, n_total - 1)


def _knn(qfeat, cand3, bounds, n):
    nblocks = n // B_Q
    nt = cand3.shape[0]
    grid_spec = pltpu.PrefetchScalarGridSpec(
        num_scalar_prefetch=1,
        grid=(nblocks,),
        in_specs=[
            pl.BlockSpec((B_Q, 8), lambda i, b: (i, 0)),
            pl.BlockSpec((nt, 8, C_T), lambda i, b: (0, 0, 0)),
        ],
        out_specs=pl.BlockSpec((B_Q, K), lambda i, b: (i, 0)),
    )
    return pl.pallas_call(
        _knn_body,
        grid_spec=grid_spec,
        out_shape=jax.ShapeDtypeStruct((n, K), jnp.int32),
    )(bounds, qfeat, cand3)


def _make_sc_gather(kn, n_rows, d, chunk):
    info = plsc.get_sparse_core_info()
    nw = info.num_cores * info.num_subcores
    per_w = kn // nw
    ni = per_w // chunk
    assert per_w % chunk == 0 and chunk % 8 == 0 and ni % 2 == 1 and ni >= 3
    mesh = plsc.VectorSubcoreMesh(core_axis_name="c", subcore_axis_name="s")

    @functools.partial(
        pl.kernel,
        mesh=mesh,
        out_type=jax.ShapeDtypeStruct((kn, d), jnp.float32),
        scratch_types=[
            pltpu.VMEM((2, chunk), jnp.int32),
            pltpu.VMEM((2, chunk, d), jnp.float32),
            pltpu.SemaphoreType.DMA,
            pltpu.SemaphoreType.DMA,
        ],
        compiler_params=pltpu.CompilerParams(use_tc_tiling_on_sc=False),
    )
    def gather_k(idx_hbm, table_hbm, out_hbm, idx_v, rows_v, sem0, sem1):
        wid = lax.axis_index("s") * info.num_cores + lax.axis_index("c")
        base = wid * per_w
        sems = (sem0, sem1)

        def fetch_and_start(g, buf):
            pltpu.sync_copy(idx_hbm.at[pl.ds(base + g * chunk, chunk)],
                            idx_v.at[buf])
            pltpu.async_copy(table_hbm.at[idx_v.at[buf]], rows_v.at[buf],
                             sems[buf])

        def finish(g, buf):
            pltpu.make_async_copy(table_hbm.at[idx_v.at[buf]],
                                  rows_v.at[buf], sems[buf]).wait()
            pltpu.sync_copy(rows_v.at[buf],
                            out_hbm.at[pl.ds(base + g * chunk, chunk)])

        fetch_and_start(0, 0)

        def pair(p, carry):
            g = 2 * p
            fetch_and_start(g + 1, 1)
            finish(g, 0)
            fetch_and_start(g + 2, 0)
            finish(g + 1, 1)
            return carry

        lax.fori_loop(0, (ni - 1) // 2, pair, 0)
        finish(ni - 1, 0)

    return gather_k


def _conv1_body(pj_ref, qf_ref, wsum_ref, wbot_ref, b1a_ref, w1b_ref,
                b1b_ref, h_ref):
    B = qf_ref.shape[0]
    pj = pj_ref[...].reshape(K * B, 16)
    a = jnp.dot(pj, wsum_ref[...], preferred_element_type=jnp.float32)
    c = jnp.dot(qf_ref[...], wbot_ref[...], preferred_element_type=jnp.float32)
    m = a.reshape(K, B, 32) - c[None, :, :] + b1a_ref[0:1, :]
    m = jnp.maximum(m, 0.0)
    m = jnp.dot(m.reshape(K * B, 32), w1b_ref[...],
                preferred_element_type=jnp.float32) + b1b_ref[0:1, :]
    m = m.reshape(K, B, 32)
    r = m[0]
    for k in range(1, K):
        r = jnp.maximum(r, m[k])
    h_ref[...] = jnp.maximum(r, 0.0)


def _conv2_body(hj_ref, pj_ref, qf_ref, wh_ref, wp16_ref, wp8_ref, b2a_ref,
                w2b_ref, b2b_ref, o_ref):
    B = qf_ref.shape[0]
    hj = hj_ref[...].reshape(K * B, 32)
    pj = pj_ref[...].reshape(K * B, 16)
    a = jnp.dot(hj, wh_ref[...], preferred_element_type=jnp.float32)
    a = a + jnp.dot(pj, wp16_ref[...], preferred_element_type=jnp.float32)
    c = jnp.dot(qf_ref[...], wp8_ref[...], preferred_element_type=jnp.float32)
    m = a.reshape(K, B, 32) - c[None, :, :] + b2a_ref[0:1, :]
    m = jnp.maximum(m, 0.0)
    m = jnp.dot(m.reshape(K * B, 32), w2b_ref[...],
                preferred_element_type=jnp.float32) + b2b_ref[0:1, :]
    m = m.reshape(K, B, 32)
    r = m[0]
    for k in range(1, K):
        r = jnp.maximum(r, m[k])
    o_ref[...] = r


def _conv1(pjg, qfeat, wsum, wbot, b1a8, w1b, b1b8, n):
    nblocks = n // B_Q
    return pl.pallas_call(
        _conv1_body,
        grid=(nblocks,),
        in_specs=[
            pl.BlockSpec((K, B_Q, 16), lambda i: (0, i, 0)),
            pl.BlockSpec((B_Q, 8), lambda i: (i, 0)),
            pl.BlockSpec((16, 32), lambda i: (0, 0)),
            pl.BlockSpec((8, 32), lambda i: (0, 0)),
            pl.BlockSpec((8, 32), lambda i: (0, 0)),
            pl.BlockSpec((32, 32), lambda i: (0, 0)),
            pl.BlockSpec((8, 32), lambda i: (0, 0)),
        ],
        out_specs=pl.BlockSpec((B_Q, 32), lambda i: (i, 0)),
        out_shape=jax.ShapeDtypeStruct((n, 32), jnp.float32),
    )(pjg, qfeat, wsum, wbot, b1a8, w1b, b1b8)


def _conv2(hjg, pjg, qfeat, wh, wp16, wp8, b2a8, w2b, b2b8, n):
    nblocks = n // B_Q
    return pl.pallas_call(
        _conv2_body,
        grid=(nblocks,),
        in_specs=[
            pl.BlockSpec((K, B_Q, 32), lambda i: (0, i, 0)),
            pl.BlockSpec((K, B_Q, 16), lambda i: (0, i, 0)),
            pl.BlockSpec((B_Q, 8), lambda i: (i, 0)),
            pl.BlockSpec((32, 32), lambda i: (0, 0)),
            pl.BlockSpec((16, 32), lambda i: (0, 0)),
            pl.BlockSpec((8, 32), lambda i: (0, 0)),
            pl.BlockSpec((8, 32), lambda i: (0, 0)),
            pl.BlockSpec((32, 32), lambda i: (0, 0)),
            pl.BlockSpec((8, 32), lambda i: (0, 0)),
        ],
        out_specs=pl.BlockSpec((B_Q, 32), lambda i: (i, 0)),
        out_shape=jax.ShapeDtypeStruct((n, 32), jnp.float32),
    )(hjg, pjg, qfeat, wh, wp16, wp8, b2a8, w2b, b2b8)


def _pad_rows(w, rows):
    return jnp.concatenate(
        [w, jnp.zeros((rows - w.shape[0], w.shape[1]), w.dtype)], axis=0)


def kernel(pos, batch, W1a, b1a, W1b, b1b, W2a, b2a, W2b, b2b):
    n = pos.shape[0]
    batch_i = batch.astype(jnp.int32)
    batchf = batch_i.astype(jnp.float32)

    qfeat = jnp.concatenate(
        [pos, batchf[:, None], jnp.zeros((n, 4), jnp.float32)], axis=1)

    n_pad = ((n + C_T - 1) // C_T) * C_T
    cand = jnp.concatenate(
        [pos.T, batchf[None, :], jnp.zeros((4, n), jnp.float32)], axis=0)
    pad_col = jnp.zeros((8, n_pad - n), jnp.float32).at[3, :].set(-1.0)
    cand = jnp.concatenate([cand, pad_col], axis=1)
    cand3 = cand.reshape(8, n_pad // C_T, C_T).transpose(1, 0, 2)

    b_first = batch_i[::B_Q]
    b_last = batch_i[B_Q - 1::B_Q]
    start = jnp.searchsorted(batch_i, b_first, side="left")
    end = jnp.searchsorted(batch_i, b_last, side="right")
    bounds = jnp.stack(
        [start // C_T, (end + C_T - 1) // C_T], axis=1).astype(jnp.int32)

    nbr = _knn(qfeat, cand3, bounds, n)
    idx_flat = nbr.T.reshape(-1)

    pos16 = jnp.concatenate([pos, jnp.zeros((n, 13), jnp.float32)], axis=1)
    gather_pos = _make_sc_gather(K * n, n, 16, 1000)
    pjg = gather_pos(idx_flat, pos16).reshape(K, n, 16)

    wsum = _pad_rows(W1a[0:3] + W1a[3:6], 16)
    wbot = _pad_rows(W1a[3:6], 8)
    b1a8 = jnp.broadcast_to(b1a[None, :], (8, 32))
    b1b8 = jnp.broadcast_to(b1b[None, :], (8, 32))
    h = _conv1(pjg, qfeat, wsum, wbot, b1a8, W1b, b1b8, n)

    gather_h = _make_sc_gather(K * n, n, 32, 1000)
    hjg = gather_h(idx_flat, h).reshape(K, n, 32)

    wh = W2a[0:32]
    wp16 = _pad_rows(W2a[32:35], 16)
    wp8 = _pad_rows(W2a[32:35], 8)
    b2a8 = jnp.broadcast_to(b2a[None, :], (8, 32))
    b2b8 = jnp.broadcast_to(b2b[None, :], (8, 32))
    return _conv2(hjg, pjg, qfeat, wh, wp16, wp8, b2a8, W2b, b2b8, n)


# speedup vs baseline: 46.9632x; 1.0008x over previous
import functools

import jax
import jax.numpy as jnp
from jax import lax
from jax.experimental import pallas as pl
from jax.experimental.pallas import tpu as pltpu
from jax.experimental.pallas import tpu_sc as plsc

K = 16
B_Q = 400
C_T = 512
BIG_F = 1.0e9


def _knn_body(bounds_ref, qf_ref, cand_ref, nbr_ref):
    i = pl.program_id(0)
    t0 = bounds_ref[i, 0]
    t1 = bounds_ref[i, 1]
    qx = qf_ref[:, 0:1]
    qy = qf_ref[:, 1:2]
    qz = qf_ref[:, 2:3]
    qb = qf_ref[:, 3:4]
    B = qf_ref.shape[0]

    def tile_step(t, carry):
        bd, bi = carry
        cand = cand_ref[t]
        cx = cand[0:1, :]
        cy = cand[1:2, :]
        cz = cand[2:3, :]
        cb = cand[3:4, :]
        dx = qx - cx
        dy = qy - cy
        dz = qz - cz
        d = (dx * dx + dy * dy) + dz * dz
        d = jnp.where(qb == cb, d, jnp.inf)
        colg = (lax.broadcasted_iota(jnp.int32, (B, C_T), 1).astype(jnp.float32)
                + t.astype(jnp.float32) * float(C_T))
        cd = jnp.concatenate([bd, d], axis=1)
        ci = jnp.concatenate([bi, colg], axis=1)
        nds, nis = [], []
        for _ in range(K):
            dmin = jnp.min(cd, axis=1, keepdims=True)
            imin = jnp.min(jnp.where(cd == dmin, ci, BIG_F), axis=1,
                           keepdims=True)
            nds.append(dmin)
            nis.append(imin)
            cd = jnp.where(ci == imin, jnp.inf, cd)
        return jnp.concatenate(nds, axis=1), jnp.concatenate(nis, axis=1)

    bd0 = jnp.full((B, K), jnp.inf, jnp.float32)
    bi0 = jnp.full((B, K), BIG_F, jnp.float32)
    _, bi = lax.fori_loop(t0, t1, tile_step, (bd0, bi0))
    n_total = nbr_ref.shape[0] * pl.num_programs(0)
    nbr_ref[...] = jnp.clip(bi.astype(jnp.int32), 0, n_total - 1)


def _knn(qfeat, cand3, bounds, n):
    nblocks = n // B_Q
    nt = cand3.shape[0]
    grid_spec = pltpu.PrefetchScalarGridSpec(
        num_scalar_prefetch=1,
        grid=(nblocks,),
        in_specs=[
            pl.BlockSpec((B_Q, 8), lambda i, b: (i, 0)),
            pl.BlockSpec((nt, 8, C_T), lambda i, b: (0, 0, 0)),
        ],
        out_specs=pl.BlockSpec((B_Q, K), lambda i, b: (i, 0)),
    )
    return pl.pallas_call(
        _knn_body,
        grid_spec=grid_spec,
        out_shape=jax.ShapeDtypeStruct((n, K), jnp.int32),
    )(bounds, qfeat, cand3)


def _make_sc_gather(kn, n_rows, d, chunk):
    info = plsc.get_sparse_core_info()
    nw = info.num_cores * info.num_subcores
    per_w = kn // nw
    ni = per_w // chunk
    assert per_w % chunk == 0 and chunk % 8 == 0 and ni % 2 == 1 and ni >= 3
    mesh = plsc.VectorSubcoreMesh(core_axis_name="c", subcore_axis_name="s")

    @functools.partial(
        pl.kernel,
        mesh=mesh,
        out_type=jax.ShapeDtypeStruct((kn, d), jnp.float32),
        scratch_types=[
            pltpu.VMEM((2, chunk), jnp.int32),
            pltpu.VMEM((2, chunk, d), jnp.float32),
            pltpu.SemaphoreType.DMA,
            pltpu.SemaphoreType.DMA,
        ],
        compiler_params=pltpu.CompilerParams(use_tc_tiling_on_sc=False),
    )
    def gather_k(idx_hbm, table_hbm, out_hbm, idx_v, rows_v, sem0, sem1):
        wid = lax.axis_index("s") * info.num_cores + lax.axis_index("c")
        base = wid * per_w
        sems = (sem0, sem1)

        def fetch_and_start(g, buf):
            pltpu.sync_copy(idx_hbm.at[pl.ds(base + g * chunk, chunk)],
                            idx_v.at[buf])
            pltpu.async_copy(table_hbm.at[idx_v.at[buf]], rows_v.at[buf],
                             sems[buf])

        def finish(g, buf):
            pltpu.make_async_copy(table_hbm.at[idx_v.at[buf]],
                                  rows_v.at[buf], sems[buf]).wait()
            pltpu.sync_copy(rows_v.at[buf],
                            out_hbm.at[pl.ds(base + g * chunk, chunk)])

        fetch_and_start(0, 0)

        def pair(p, carry):
            g = 2 * p
            fetch_and_start(g + 1, 1)
            finish(g, 0)
            fetch_and_start(g + 2, 0)
            finish(g + 1, 1)
            return carry

        lax.fori_loop(0, (ni - 1) // 2, pair, 0)
        finish(ni - 1, 0)

    return gather_k


def _conv1_body(pj_ref, qf_ref, wsum_ref, wbot_ref, b1a_ref, w1b_ref,
                b1b_ref, h_ref):
    B = qf_ref.shape[0]
    pj = pj_ref[...].reshape(K * B, 16)
    a = jnp.dot(pj, wsum_ref[...], preferred_element_type=jnp.float32)
    c = jnp.dot(qf_ref[...], wbot_ref[...], preferred_element_type=jnp.float32)
    c = c - b1a_ref[0:1, :]
    m = jnp.maximum(a.reshape(K, B, 32) - c[None, :, :], 0.0)
    m = jnp.dot(m.reshape(K * B, 32), w1b_ref[...],
                preferred_element_type=jnp.float32)
    m = m.reshape(K, B, 32)
    r = m[0]
    for k in range(1, K):
        r = jnp.maximum(r, m[k])
    h_ref[...] = jnp.maximum(r + b1b_ref[0:1, :], 0.0)


def _conv2_body(hj_ref, pj_ref, qf_ref, wh_ref, wp16_ref, wp8_ref, b2a_ref,
                w2b_ref, b2b_ref, o_ref):
    B = qf_ref.shape[0]
    hj = hj_ref[...].reshape(K * B, 32)
    pj = pj_ref[...].reshape(K * B, 16)
    a = jnp.dot(hj, wh_ref[...], preferred_element_type=jnp.float32)
    a = a + jnp.dot(pj, wp16_ref[...], preferred_element_type=jnp.float32)
    c = jnp.dot(qf_ref[...], wp8_ref[...], preferred_element_type=jnp.float32)
    c = c - b2a_ref[0:1, :]
    m = jnp.maximum(a.reshape(K, B, 32) - c[None, :, :], 0.0)
    m = jnp.dot(m.reshape(K * B, 32), w2b_ref[...],
                preferred_element_type=jnp.float32)
    m = m.reshape(K, B, 32)
    r = m[0]
    for k in range(1, K):
        r = jnp.maximum(r, m[k])
    o_ref[...] = r + b2b_ref[0:1, :]


def _conv1(pjg, qfeat, wsum, wbot, b1a8, w1b, b1b8, n):
    nblocks = n // B_Q
    return pl.pallas_call(
        _conv1_body,
        grid=(nblocks,),
        in_specs=[
            pl.BlockSpec((K, B_Q, 16), lambda i: (0, i, 0)),
            pl.BlockSpec((B_Q, 8), lambda i: (i, 0)),
            pl.BlockSpec((16, 32), lambda i: (0, 0)),
            pl.BlockSpec((8, 32), lambda i: (0, 0)),
            pl.BlockSpec((8, 32), lambda i: (0, 0)),
            pl.BlockSpec((32, 32), lambda i: (0, 0)),
            pl.BlockSpec((8, 32), lambda i: (0, 0)),
        ],
        out_specs=pl.BlockSpec((B_Q, 32), lambda i: (i, 0)),
        out_shape=jax.ShapeDtypeStruct((n, 32), jnp.float32),
    )(pjg, qfeat, wsum, wbot, b1a8, w1b, b1b8)


def _conv2(hjg, pjg, qfeat, wh, wp16, wp8, b2a8, w2b, b2b8, n):
    nblocks = n // B_Q
    return pl.pallas_call(
        _conv2_body,
        grid=(nblocks,),
        in_specs=[
            pl.BlockSpec((K, B_Q, 32), lambda i: (0, i, 0)),
            pl.BlockSpec((K, B_Q, 16), lambda i: (0, i, 0)),
            pl.BlockSpec((B_Q, 8), lambda i: (i, 0)),
            pl.BlockSpec((32, 32), lambda i: (0, 0)),
            pl.BlockSpec((16, 32), lambda i: (0, 0)),
            pl.BlockSpec((8, 32), lambda i: (0, 0)),
            pl.BlockSpec((8, 32), lambda i: (0, 0)),
            pl.BlockSpec((32, 32), lambda i: (0, 0)),
            pl.BlockSpec((8, 32), lambda i: (0, 0)),
        ],
        out_specs=pl.BlockSpec((B_Q, 32), lambda i: (i, 0)),
        out_shape=jax.ShapeDtypeStruct((n, 32), jnp.float32),
    )(hjg, pjg, qfeat, wh, wp16, wp8, b2a8, w2b, b2b8)


def _pad_rows(w, rows):
    return jnp.concatenate(
        [w, jnp.zeros((rows - w.shape[0], w.shape[1]), w.dtype)], axis=0)


def kernel(pos, batch, W1a, b1a, W1b, b1b, W2a, b2a, W2b, b2b):
    n = pos.shape[0]
    batch_i = batch.astype(jnp.int32)
    batchf = batch_i.astype(jnp.float32)

    qfeat = jnp.concatenate(
        [pos, batchf[:, None], jnp.zeros((n, 4), jnp.float32)], axis=1)

    n_pad = ((n + C_T - 1) // C_T) * C_T
    cand = jnp.concatenate(
        [pos.T, batchf[None, :], jnp.zeros((4, n), jnp.float32)], axis=0)
    pad_col = jnp.zeros((8, n_pad - n), jnp.float32).at[3, :].set(-1.0)
    cand = jnp.concatenate([cand, pad_col], axis=1)
    cand3 = cand.reshape(8, n_pad // C_T, C_T).transpose(1, 0, 2)

    b_first = batch_i[::B_Q]
    b_last = batch_i[B_Q - 1::B_Q]
    start = jnp.searchsorted(batch_i, b_first, side="left")
    end = jnp.searchsorted(batch_i, b_last, side="right")
    bounds = jnp.stack(
        [start // C_T, (end + C_T - 1) // C_T], axis=1).astype(jnp.int32)

    nbr = _knn(qfeat, cand3, bounds, n)
    idx_flat = nbr.T.reshape(-1)

    pos16 = jnp.concatenate([pos, jnp.zeros((n, 13), jnp.float32)], axis=1)
    gather_pos = _make_sc_gather(K * n, n, 16, 1000)
    pjg = gather_pos(idx_flat, pos16).reshape(K, n, 16)

    wsum = _pad_rows(W1a[0:3] + W1a[3:6], 16)
    wbot = _pad_rows(W1a[3:6], 8)
    b1a8 = jnp.broadcast_to(b1a[None, :], (8, 32))
    b1b8 = jnp.broadcast_to(b1b[None, :], (8, 32))
    h = _conv1(pjg, qfeat, wsum, wbot, b1a8, W1b, b1b8, n)

    gather_h = _make_sc_gather(K * n, n, 32, 1000)
    hjg = gather_h(idx_flat, h).reshape(K, n, 32)

    wh = W2a[0:32]
    wp16 = _pad_rows(W2a[32:35], 16)
    wp8 = _pad_rows(W2a[32:35], 8)
    b2a8 = jnp.broadcast_to(b2a[None, :], (8, 32))
    b2b8 = jnp.broadcast_to(b2b[None, :], (8, 32))
    return _conv2(hjg, pjg, qfeat, wh, wp16, wp8, b2a8, W2b, b2b8, n)
